# baseline (speedup 1.0000x reference)
import jax
import jax.numpy as jnp
from jax.experimental import pallas as pl
from jax.experimental.pallas import tpu as pltpu

_B, _N, _Q = 2, 16384, 4096
_CIN, _LAT, _COUT = 3, 64, 2
_NSUP, _K = 3000, 16
_MSUP = 3072
_TN = 1024
_TQ = 256
_BIG = 1e30
_FAR = 1e9


def _down_kernel(xd_ref, w1_ref, b1_ref, w2_ref, b2_ref, out_ref):
    x = xd_ref[0]
    h = jnp.maximum(
        jnp.dot(x, w1_ref[...], preferred_element_type=jnp.float32) + b1_ref[...], 0.0)
    out_ref[0] = jnp.dot(h, w2_ref[...], preferred_element_type=jnp.float32) + b2_ref[...]


def _main_kernel(xt_ref, pt_ref, pd_ref, latd_ref,
                 w1_ref, b1_ref, w2_ref, b2_ref,
                 mw1a_ref, mw1b_ref, mb1_ref, mw2_ref, mb2_ref, mw3_ref, mb3_ref,
                 out_ref):
    x = xt_ref[0]
    p = pt_ref[0]
    pd = pd_ref[0]
    latd = latd_ref[0]

    h = jnp.maximum(
        jnp.dot(x, w1_ref[...], preferred_element_type=jnp.float32) + b1_ref[...], 0.0)
    lat = jnp.dot(h, w2_ref[...], preferred_element_type=jnp.float32) + b2_ref[...]

    rr = jnp.sum(pd * pd, axis=0, keepdims=True)
    qq = jnp.sum(p * p, axis=1, keepdims=True)
    d2 = (qq - 2.0 * jnp.dot(p, pd)) + rr
    m = jnp.min(d2, axis=1, keepdims=True)
    mask = (d2 <= m).astype(jnp.float32)
    cnt = jnp.sum(mask, axis=1, keepdims=True)
    up = jnp.dot(mask, latd, preferred_element_type=jnp.float32) / cnt

    hm = jnp.maximum(
        jnp.dot(lat, mw1a_ref[...], preferred_element_type=jnp.float32)
        + jnp.dot(up, mw1b_ref[...], preferred_element_type=jnp.float32)
        + mb1_ref[...], 0.0)
    hm = jnp.maximum(
        jnp.dot(hm, mw2_ref[...], preferred_element_type=jnp.float32) + mb2_ref[...], 0.0)
    out_ref[0] = jnp.dot(hm, mw3_ref[...], preferred_element_type=jnp.float32) + mb3_ref[...]


def _query_kernel(qt_ref, p3_ref, latm_ref, pw_ref, pb_ref, out_ref):
    q = qt_ref[0]
    p3 = p3_ref[0]
    latm = latm_ref[0]

    rr = jnp.sum(p3 * p3, axis=0, keepdims=True)
    qq = jnp.sum(q * q, axis=1, keepdims=True)
    d2 = (qq - 2.0 * jnp.dot(q, p3)) + rr

    def body(_, d2c):
        m = jnp.min(d2c, axis=1, keepdims=True)
        return jnp.where(d2c <= m, _BIG, d2c)

    d2f = jax.lax.fori_loop(0, _K, body, d2, unroll=16)
    tot = (d2f >= 0.5 * _BIG).astype(jnp.float32)
    cnt = jnp.sum(tot, axis=1, keepdims=True)
    feat = jnp.dot(tot, latm, preferred_element_type=jnp.float32) / cnt
    out_ref[0] = jnp.dot(feat, pw_ref[...], preferred_element_type=jnp.float32) + pb_ref[...]


def kernel(x, pos, pos_non_manifold, bb_w1, bb_b1, bb_w2, bb_b2,
           m_w1, m_b1, m_w2, m_b2, m_w3, m_b3, p_w, p_b):
    f32 = jnp.float32
    xT = jnp.transpose(x, (0, 2, 1))
    posT = jnp.transpose(pos, (0, 2, 1))
    qT = jnp.transpose(pos_non_manifold, (0, 2, 1))
    xdT = jnp.pad(xT[:, :_NSUP, :], ((0, 0), (0, _MSUP - _NSUP), (0, 0)))
    pd3 = jnp.pad(pos[:, :, :_NSUP], ((0, 0), (0, 0), (0, _MSUP - _NSUP)),
                  constant_values=_FAR)

    w1T = bb_w1.T.astype(f32)
    w2T = bb_w2.T.astype(f32)
    b1r = bb_b1.reshape(1, -1)
    b2r = bb_b2.reshape(1, -1)
    mw1aT = m_w1[:, :_LAT].T.astype(f32)
    mw1bT = m_w1[:, _LAT:].T.astype(f32)
    mw2T = m_w2.T.astype(f32)
    mw3T = m_w3.T.astype(f32)
    mb1r = m_b1.reshape(1, -1)
    mb2r = m_b2.reshape(1, -1)
    mb3r = m_b3.reshape(1, -1)
    pwT = p_w.T.astype(f32)
    pbr = p_b.reshape(1, -1)

    def full2d(arr):
        return pl.BlockSpec(arr.shape, lambda *_: (0,) * arr.ndim)

    latd = pl.pallas_call(
        _down_kernel,
        grid=(_B,),
        in_specs=[
            pl.BlockSpec((1, _MSUP, _CIN), lambda b: (b, 0, 0)),
            full2d(w1T), full2d(b1r), full2d(w2T), full2d(b2r),
        ],
        out_specs=pl.BlockSpec((1, _MSUP, _LAT), lambda b: (b, 0, 0)),
        out_shape=jax.ShapeDtypeStruct((_B, _MSUP, _LAT), f32),
    )(xdT, w1T, b1r, w2T, b2r)

    latm = pl.pallas_call(
        _main_kernel,
        grid=(_B, _N // _TN),
        in_specs=[
            pl.BlockSpec((1, _TN, _CIN), lambda b, i: (b, i, 0)),
            pl.BlockSpec((1, _TN, _CIN), lambda b, i: (b, i, 0)),
            pl.BlockSpec((1, 3, _MSUP), lambda b, i: (b, 0, 0)),
            pl.BlockSpec((1, _MSUP, _LAT), lambda b, i: (b, 0, 0)),
            full2d(w1T), full2d(b1r), full2d(w2T), full2d(b2r),
            full2d(mw1aT), full2d(mw1bT), full2d(mb1r),
            full2d(mw2T), full2d(mb2r), full2d(mw3T), full2d(mb3r),
        ],
        out_specs=pl.BlockSpec((1, _TN, _LAT), lambda b, i: (b, i, 0)),
        out_shape=jax.ShapeDtypeStruct((_B, _N, _LAT), f32),
        compiler_params=pltpu.CompilerParams(vmem_limit_bytes=100 * 1024 * 1024),
    )(xT, posT, pd3, latd,
      w1T, b1r, w2T, b2r, mw1aT, mw1bT, mb1r, mw2T, mb2r, mw3T, mb3r)

    outT = pl.pallas_call(
        _query_kernel,
        grid=(_B, _Q // _TQ),
        in_specs=[
            pl.BlockSpec((1, _TQ, _CIN), lambda b, i: (b, i, 0)),
            pl.BlockSpec((1, 3, _N), lambda b, i: (b, 0, 0)),
            pl.BlockSpec((1, _N, _LAT), lambda b, i: (b, 0, 0)),
            full2d(pwT), full2d(pbr),
        ],
        out_specs=pl.BlockSpec((1, _TQ, _COUT), lambda b, i: (b, i, 0)),
        out_shape=jax.ShapeDtypeStruct((_B, _Q, _COUT), f32),
        compiler_params=pltpu.CompilerParams(vmem_limit_bytes=63 * 1024 * 1024),
    )(qT, pos, latm, pwT, pbr)

    return jnp.transpose(outT, (0, 2, 1))


# speedup vs baseline: 1.2678x; 1.2678x over previous
import jax
import jax.numpy as jnp
from jax.experimental import pallas as pl
from jax.experimental.pallas import tpu as pltpu

_B, _N, _Q = 2, 16384, 4096
_CIN, _LAT, _COUT = 3, 64, 2
_NSUP, _K = 3000, 16
_MSUP = 3072
_TN = 2048
_TQ = 128
_BIG = 1e30
_FAR = 1e9


def _down_kernel(xd_ref, w1_ref, b1_ref, w2_ref, b2_ref, out_ref):
    x = xd_ref[0]
    h = jnp.maximum(
        jnp.dot(x, w1_ref[...], preferred_element_type=jnp.float32) + b1_ref[...], 0.0)
    out_ref[0] = jnp.dot(h, w2_ref[...], preferred_element_type=jnp.float32) + b2_ref[...]


def _main_kernel(xt_ref, pt_ref, pd_ref, latd_ref,
                 w1_ref, b1_ref, w2_ref, b2_ref,
                 mw1a_ref, mw1b_ref, mb1_ref, mw2_ref, mb2_ref, mw3_ref, mb3_ref,
                 out_ref):
    x = xt_ref[0]
    p = pt_ref[0]
    pd = pd_ref[0]
    latd = latd_ref[0]

    h = jnp.maximum(
        jnp.dot(x, w1_ref[...], preferred_element_type=jnp.float32) + b1_ref[...], 0.0)
    lat = jnp.dot(h, w2_ref[...], preferred_element_type=jnp.float32) + b2_ref[...]

    rr = jnp.sum(pd * pd, axis=0, keepdims=True)
    qq = jnp.sum(p * p, axis=1, keepdims=True)
    d2 = (qq - 2.0 * jnp.dot(p, pd)) + rr
    m = jnp.min(d2, axis=1, keepdims=True)
    mask = (d2 <= m).astype(jnp.float32)
    cnt = jnp.sum(mask, axis=1, keepdims=True)
    up = jnp.dot(mask, latd, preferred_element_type=jnp.float32) / cnt

    hm = jnp.maximum(
        jnp.dot(lat, mw1a_ref[...], preferred_element_type=jnp.float32)
        + jnp.dot(up, mw1b_ref[...], preferred_element_type=jnp.float32)
        + mb1_ref[...], 0.0)
    hm = jnp.maximum(
        jnp.dot(hm, mw2_ref[...], preferred_element_type=jnp.float32) + mb2_ref[...], 0.0)
    out_ref[0] = jnp.dot(hm, mw3_ref[...], preferred_element_type=jnp.float32) + mb3_ref[...]


def _query_kernel(qt_ref, p3_ref, latm_ref, pw_ref, pb_ref, out_ref):
    q = qt_ref[0]
    p3 = p3_ref[0]
    latm = latm_ref[0]

    rr = jnp.sum(p3 * p3, axis=0, keepdims=True)
    qq = jnp.sum(q * q, axis=1, keepdims=True)
    d2 = (qq - 2.0 * jnp.dot(q, p3)) + rr

    def body(_, d2c):
        m = jnp.min(d2c, axis=1, keepdims=True)
        return jnp.where(d2c <= m, _BIG, d2c)

    d2f = jax.lax.fori_loop(0, _K, body, d2, unroll=16)
    tot = (d2f >= 0.5 * _BIG).astype(jnp.float32)
    cnt = jnp.sum(tot, axis=1, keepdims=True)
    feat = jnp.dot(tot, latm, preferred_element_type=jnp.float32) / cnt
    out_ref[0] = jnp.dot(feat, pw_ref[...], preferred_element_type=jnp.float32) + pb_ref[...]


def kernel(x, pos, pos_non_manifold, bb_w1, bb_b1, bb_w2, bb_b2,
           m_w1, m_b1, m_w2, m_b2, m_w3, m_b3, p_w, p_b):
    f32 = jnp.float32
    xT = jnp.transpose(x, (0, 2, 1))
    posT = jnp.transpose(pos, (0, 2, 1))
    qT = jnp.transpose(pos_non_manifold, (0, 2, 1))
    xdT = jnp.pad(xT[:, :_NSUP, :], ((0, 0), (0, _MSUP - _NSUP), (0, 0)))
    pd3 = jnp.pad(pos[:, :, :_NSUP], ((0, 0), (0, 0), (0, _MSUP - _NSUP)),
                  constant_values=_FAR)

    w1T = bb_w1.T.astype(f32)
    w2T = bb_w2.T.astype(f32)
    b1r = bb_b1.reshape(1, -1)
    b2r = bb_b2.reshape(1, -1)
    mw1aT = m_w1[:, :_LAT].T.astype(f32)
    mw1bT = m_w1[:, _LAT:].T.astype(f32)
    mw2T = m_w2.T.astype(f32)
    mw3T = m_w3.T.astype(f32)
    mb1r = m_b1.reshape(1, -1)
    mb2r = m_b2.reshape(1, -1)
    mb3r = m_b3.reshape(1, -1)
    pwT = p_w.T.astype(f32)
    pbr = p_b.reshape(1, -1)

    def full2d(arr):
        return pl.BlockSpec(arr.shape, lambda *_: (0,) * arr.ndim)

    latd = pl.pallas_call(
        _down_kernel,
        grid=(_B,),
        in_specs=[
            pl.BlockSpec((1, _MSUP, _CIN), lambda b: (b, 0, 0)),
            full2d(w1T), full2d(b1r), full2d(w2T), full2d(b2r),
        ],
        out_specs=pl.BlockSpec((1, _MSUP, _LAT), lambda b: (b, 0, 0)),
        out_shape=jax.ShapeDtypeStruct((_B, _MSUP, _LAT), f32),
    )(xdT, w1T, b1r, w2T, b2r)

    latm = pl.pallas_call(
        _main_kernel,
        grid=(_B, _N // _TN),
        in_specs=[
            pl.BlockSpec((1, _TN, _CIN), lambda b, i: (b, i, 0)),
            pl.BlockSpec((1, _TN, _CIN), lambda b, i: (b, i, 0)),
            pl.BlockSpec((1, 3, _MSUP), lambda b, i: (b, 0, 0)),
            pl.BlockSpec((1, _MSUP, _LAT), lambda b, i: (b, 0, 0)),
            full2d(w1T), full2d(b1r), full2d(w2T), full2d(b2r),
            full2d(mw1aT), full2d(mw1bT), full2d(mb1r),
            full2d(mw2T), full2d(mb2r), full2d(mw3T), full2d(mb3r),
        ],
        out_specs=pl.BlockSpec((1, _TN, _LAT), lambda b, i: (b, i, 0)),
        out_shape=jax.ShapeDtypeStruct((_B, _N, _LAT), f32),
        compiler_params=pltpu.CompilerParams(vmem_limit_bytes=100 * 1024 * 1024),
    )(xT, posT, pd3, latd,
      w1T, b1r, w2T, b2r, mw1aT, mw1bT, mb1r, mw2T, mb2r, mw3T, mb3r)

    outT = pl.pallas_call(
        _query_kernel,
        grid=(_B, _Q // _TQ),
        in_specs=[
            pl.BlockSpec((1, _TQ, _CIN), lambda b, i: (b, i, 0)),
            pl.BlockSpec((1, 3, _N), lambda b, i: (b, 0, 0)),
            pl.BlockSpec((1, _N, _LAT), lambda b, i: (b, 0, 0)),
            full2d(pwT), full2d(pbr),
        ],
        out_specs=pl.BlockSpec((1, _TQ, _COUT), lambda b, i: (b, i, 0)),
        out_shape=jax.ShapeDtypeStruct((_B, _Q, _COUT), f32),
        compiler_params=pltpu.CompilerParams(vmem_limit_bytes=100 * 1024 * 1024),
    )(qT, pos, latm, pwT, pbr)

    return jnp.transpose(outT, (0, 2, 1))
